# Initial kernel scaffold; baseline (speedup 1.0000x reference)
import jax
import jax.numpy as jnp
from jax.experimental import pallas as pl


def kernel(h_bot, c_bot, h_buf, c_buf, Wl, Wr, b, bot_froms, prev_froms):
    raise NotImplementedError("write your pallas kernel here")


# speedup vs baseline: 7.2856x; 7.2856x over previous
import functools

import jax
import jax.numpy as jnp
from jax import lax
from jax.experimental import pallas as pl
from jax.experimental.pallas import tpu as pltpu
from jax.experimental.pallas import tpu_sc as plsc

D = 128
_LB = 60000
_LP = 40000
_T = _LB + _LP

_CH = 400
_NB_CHUNKS = _LB // _CH
_NP_CHUNKS = _LP // _CH
_NW = 32
_CHUNKS = 256
_TPAD = _CHUNKS * _CH
_CPW = _CHUNKS // _NW

_BT = 1000


def _sc_gather(h_bot, c_bot, h_buf, c_buf, idx_l, idx_r):
  mesh = plsc.VectorSubcoreMesh(core_axis_name="c", subcore_axis_name="s")

  @functools.partial(
      pl.kernel, mesh=mesh,
      out_type=[jax.ShapeDtypeStruct((_TPAD, D), jnp.float32)] * 4,
      scratch_types=[
          pltpu.VMEM((_CH,), jnp.int32),
          pltpu.VMEM((_CH, D), jnp.float32),
          pltpu.VMEM((_CH, D), jnp.float32),
          pltpu.SemaphoreType.DMA,
          pltpu.SemaphoreType.DMA,
      ],
  )
  def k(hb, cb, hf, cf, idx_l_hbm, idx_r_hbm, hl_o, cl_o, hr_o, cr_o,
        idx_v, h_v, c_v, sem_h, sem_c):
    wid = lax.axis_index("s") * 2 + lax.axis_index("c")

    def chunk(i, carry):
      kid = wid * _CPW + i
      base = kid * _CH
      is_buf = jnp.logical_and(kid >= _NB_CHUNKS,
                               kid < _NB_CHUNKS + _NP_CHUNKS)
      for idx_hbm, h_o, c_o in ((idx_l_hbm, hl_o, cl_o),
                                (idx_r_hbm, hr_o, cr_o)):
        pltpu.sync_copy(idx_hbm.at[pl.ds(base, _CH)], idx_v)

        @pl.when(is_buf)
        def _():
          a = pltpu.async_copy(hf.at[idx_v], h_v, sem_h)
          b = pltpu.async_copy(cf.at[idx_v], c_v, sem_c)
          a.wait()
          b.wait()

        @pl.when(jnp.logical_not(is_buf))
        def _():
          a = pltpu.async_copy(hb.at[idx_v], h_v, sem_h)
          b = pltpu.async_copy(cb.at[idx_v], c_v, sem_c)
          a.wait()
          b.wait()

        pltpu.sync_copy(h_v, h_o.at[pl.ds(base, _CH)])
        pltpu.sync_copy(c_v, c_o.at[pl.ds(base, _CH)])
      return carry

    lax.fori_loop(0, _CPW, chunk, 0)

  return k(h_bot, c_bot, h_buf, c_buf, idx_l, idx_r)


def _tc_cell_body(hl_r, hr_r, cl_r, cr_r, wl_r, wr_r, b_r, h_o, c_o):
  g = jnp.dot(hl_r[...], wl_r[...], preferred_element_type=jnp.float32)
  g = g + jnp.dot(hr_r[...], wr_r[...], preferred_element_type=jnp.float32)
  g = g + b_r[0:1, :]
  i = jax.nn.sigmoid(g[:, 0:D])
  o = jax.nn.sigmoid(g[:, D:2 * D])
  u = jnp.tanh(g[:, 2 * D:3 * D])
  fl = jax.nn.sigmoid(g[:, 3 * D:4 * D])
  fr = jax.nn.sigmoid(g[:, 4 * D:5 * D])
  c = i * u + fl * cl_r[...] + fr * cr_r[...]
  h_o[...] = o * jnp.tanh(c)
  c_o[...] = c


def _tc_cell(hl, hr, cl, cr, Wl, Wr, b2d):
  row = pl.BlockSpec((_BT, D), lambda i: (i, 0))
  wspec = pl.BlockSpec((D, 5 * D), lambda i: (0, 0))
  bspec = pl.BlockSpec((8, 5 * D), lambda i: (0, 0))
  return pl.pallas_call(
      _tc_cell_body,
      grid=(_T // _BT,),
      in_specs=[row, row, row, row, wspec, wspec, bspec],
      out_specs=[row, row],
      out_shape=[jax.ShapeDtypeStruct((_T, D), jnp.float32)] * 2,
  )(hl, hr, cl, cr, Wl, Wr, b2d)


def kernel(h_bot, c_bot, h_buf, c_buf, Wl, Wr, b, bot_froms, prev_froms):
  idx_all = jnp.concatenate([
      jnp.asarray(bot_froms, jnp.int32),
      jnp.asarray(prev_froms, jnp.int32),
      jnp.zeros((2, _TPAD - _T), jnp.int32),
  ], axis=1)
  hl, cl, hr, cr = _sc_gather(h_bot, c_bot, h_buf, c_buf,
                              idx_all[0], idx_all[1])
  b2d = jnp.broadcast_to(b.astype(jnp.float32), (8, 5 * D))
  h, c = _tc_cell(hl, hr, cl, cr, Wl, Wr, b2d)
  return (h, c)


# speedup vs baseline: 12.9784x; 1.7814x over previous
import functools

import jax
import jax.numpy as jnp
from jax import lax
from jax.experimental import pallas as pl
from jax.experimental.pallas import tpu as pltpu
from jax.experimental.pallas import tpu_sc as plsc

D = 128
_LB = 60000
_LP = 40000
_T = _LB + _LP

_CH = 400
_NB_CHUNKS = _LB // _CH
_NP_CHUNKS = _LP // _CH
_NW = 32
_CHUNKS = 256
_TPAD = _CHUNKS * _CH
_CPW = _CHUNKS // _NW

_BT = 1000


def _sc_gather(h_bot, c_bot, h_buf, c_buf, idx_l, idx_r):
  mesh = plsc.VectorSubcoreMesh(core_axis_name="c", subcore_axis_name="s")

  @functools.partial(
      pl.kernel, mesh=mesh,
      out_type=[jax.ShapeDtypeStruct((_TPAD, D), jnp.float32)] * 4,
      scratch_types=[
          pltpu.VMEM((_CH,), jnp.int32),
          pltpu.VMEM((_CH, D), jnp.float32),
          pltpu.VMEM((_CH, D), jnp.float32),
          pltpu.SemaphoreType.DMA,
          pltpu.SemaphoreType.DMA,
      ],
  )
  def k(hb, cb, hf, cf, idx_l_hbm, idx_r_hbm, hl_o, cl_o, hr_o, cr_o,
        idx_v, h_v, c_v, sem_h, sem_c):
    wid = lax.axis_index("s") * 2 + lax.axis_index("c")

    def chunk(i, carry):
      kid = i * _NW + wid
      base = kid * _CH
      is_buf = jnp.logical_and(kid >= _NB_CHUNKS,
                               kid < _NB_CHUNKS + _NP_CHUNKS)
      for idx_hbm, h_o, c_o in ((idx_l_hbm, hl_o, cl_o),
                                (idx_r_hbm, hr_o, cr_o)):
        pltpu.sync_copy(idx_hbm.at[pl.ds(base, _CH)], idx_v)

        @pl.when(is_buf)
        def _():
          a = pltpu.async_copy(hf.at[idx_v], h_v, sem_h)
          b = pltpu.async_copy(cf.at[idx_v], c_v, sem_c)
          a.wait()
          b.wait()

        @pl.when(jnp.logical_not(is_buf))
        def _():
          a = pltpu.async_copy(hb.at[idx_v], h_v, sem_h)
          b = pltpu.async_copy(cb.at[idx_v], c_v, sem_c)
          a.wait()
          b.wait()

        pltpu.sync_copy(h_v, h_o.at[pl.ds(base, _CH)])
        pltpu.sync_copy(c_v, c_o.at[pl.ds(base, _CH)])
      return carry

    lax.fori_loop(0, _CPW, chunk, 0)

  return k(h_bot, c_bot, h_buf, c_buf, idx_l, idx_r)


def _tc_cell_body(hl_r, hr_r, cl_r, cr_r, wl_r, wr_r, b_r, h_o, c_o):
  g = jnp.dot(hl_r[...], wl_r[...], preferred_element_type=jnp.float32)
  g = g + jnp.dot(hr_r[...], wr_r[...], preferred_element_type=jnp.float32)
  g = g + b_r[0:1, :]
  i = jax.nn.sigmoid(g[:, 0:D])
  o = jax.nn.sigmoid(g[:, D:2 * D])
  u = jnp.tanh(g[:, 2 * D:3 * D])
  fl = jax.nn.sigmoid(g[:, 3 * D:4 * D])
  fr = jax.nn.sigmoid(g[:, 4 * D:5 * D])
  c = i * u + fl * cl_r[...] + fr * cr_r[...]
  h_o[...] = o * jnp.tanh(c)
  c_o[...] = c


def _tc_cell(hl, hr, cl, cr, Wl, Wr, b2d):
  row = pl.BlockSpec((_BT, D), lambda i: (i, 0))
  wspec = pl.BlockSpec((D, 5 * D), lambda i: (0, 0))
  bspec = pl.BlockSpec((8, 5 * D), lambda i: (0, 0))
  return pl.pallas_call(
      _tc_cell_body,
      grid=(_T // _BT,),
      in_specs=[row, row, row, row, wspec, wspec, bspec],
      out_specs=[row, row],
      out_shape=[jax.ShapeDtypeStruct((_T, D), jnp.float32)] * 2,
  )(hl, hr, cl, cr, Wl, Wr, b2d)


def kernel(h_bot, c_bot, h_buf, c_buf, Wl, Wr, b, bot_froms, prev_froms):
  idx_all = jnp.concatenate([
      jnp.asarray(bot_froms, jnp.int32),
      jnp.asarray(prev_froms, jnp.int32),
      jnp.broadcast_to(jnp.arange(_TPAD - _T, dtype=jnp.int32),
                       (2, _TPAD - _T)),
  ], axis=1)
  hl, cl, hr, cr = _sc_gather(h_bot, c_bot, h_buf, c_buf,
                              idx_all[0], idx_all[1])
  b2d = jnp.broadcast_to(b.astype(jnp.float32), (8, 5 * D))
  h, c = _tc_cell(hl, hr, cl, cr, Wl, Wr, b2d)
  return (h, c)


# speedup vs baseline: 14.9957x; 1.1554x over previous
import functools

import jax
import jax.numpy as jnp
from jax import lax
from jax.experimental import pallas as pl
from jax.experimental.pallas import tpu as pltpu
from jax.experimental.pallas import tpu_sc as plsc

D = 128
_LB = 60000
_LP = 40000
_T = _LB + _LP

_CH = 400
_NB_CHUNKS = _LB // _CH
_NP_CHUNKS = _LP // _CH
_NW = 32
_CHUNKS = 256
_TPAD = _CHUNKS * _CH

_NSLAB = 4
_CPS = _CHUNKS // _NSLAB
_CPWS = _CPS // _NW
_SROWS = _CPS * _CH

_BT = 800
_BPS = _SROWS // _BT


def _sc_gather_slab(slab, h_bot, c_bot, h_buf, c_buf, idx_l, idx_r):
  mesh = plsc.VectorSubcoreMesh(core_axis_name="c", subcore_axis_name="s")

  @functools.partial(
      pl.kernel, mesh=mesh,
      out_type=[jax.ShapeDtypeStruct((_SROWS, D), jnp.float32)] * 4,
      scratch_types=[
          pltpu.VMEM((_CH,), jnp.int32),
          pltpu.VMEM((_CH, D), jnp.float32),
          pltpu.VMEM((_CH, D), jnp.float32),
          pltpu.SemaphoreType.DMA,
          pltpu.SemaphoreType.DMA,
      ],
  )
  def k(hb, cb, hf, cf, idx_l_hbm, idx_r_hbm, hl_o, cl_o, hr_o, cr_o,
        idx_v, h_v, c_v, sem_h, sem_c):
    wid = lax.axis_index("s") * 2 + lax.axis_index("c")

    def chunk(i, carry):
      lk = i * _NW + wid
      kid = slab * _CPS + lk
      base = lk * _CH
      is_buf = jnp.logical_and(kid >= _NB_CHUNKS,
                               kid < _NB_CHUNKS + _NP_CHUNKS)
      for idx_hbm, h_o, c_o in ((idx_l_hbm, hl_o, cl_o),
                                (idx_r_hbm, hr_o, cr_o)):
        pltpu.sync_copy(idx_hbm.at[pl.ds(base, _CH)], idx_v)

        @pl.when(is_buf)
        def _():
          a = pltpu.async_copy(hf.at[idx_v], h_v, sem_h)
          b = pltpu.async_copy(cf.at[idx_v], c_v, sem_c)
          a.wait()
          b.wait()

        @pl.when(jnp.logical_not(is_buf))
        def _():
          a = pltpu.async_copy(hb.at[idx_v], h_v, sem_h)
          b = pltpu.async_copy(cb.at[idx_v], c_v, sem_c)
          a.wait()
          b.wait()

        pltpu.sync_copy(h_v, h_o.at[pl.ds(base, _CH)])
        pltpu.sync_copy(c_v, c_o.at[pl.ds(base, _CH)])
      return carry

    lax.fori_loop(0, _CPWS, chunk, 0)

  return k(h_bot, c_bot, h_buf, c_buf, idx_l, idx_r)


def _cell_math(hl_r, hr_r, cl_r, cr_r, wl_r, wr_r, b_r, h_o, c_o):
  g = jnp.dot(hl_r[...], wl_r[...], preferred_element_type=jnp.float32)
  g = g + jnp.dot(hr_r[...], wr_r[...], preferred_element_type=jnp.float32)
  g = g + b_r[0:1, :]
  i = jax.nn.sigmoid(g[:, 0:D])
  o = jax.nn.sigmoid(g[:, D:2 * D])
  u = jnp.tanh(g[:, 2 * D:3 * D])
  fl = jax.nn.sigmoid(g[:, 3 * D:4 * D])
  fr = jax.nn.sigmoid(g[:, 4 * D:5 * D])
  c = i * u + fl * cl_r[...] + fr * cr_r[...]
  h_o[...] = o * jnp.tanh(c)
  c_o[...] = c


def _tc_body_alias(hp, cp, hl_r, hr_r, cl_r, cr_r, wl_r, wr_r, b_r, h_o, c_o):
  _cell_math(hl_r, hr_r, cl_r, cr_r, wl_r, wr_r, b_r, h_o, c_o)


def _tc_slab(slab, nblk, h_prev, c_prev, hl, hr, cl, cr, Wl, Wr, b2d):
  row_in = pl.BlockSpec((_BT, D), lambda i: (i, 0))
  row_out = pl.BlockSpec((_BT, D), lambda i, _s=slab: (_s * _BPS + i, 0))
  wspec = pl.BlockSpec((D, 5 * D), lambda i: (0, 0))
  bspec = pl.BlockSpec((8, 5 * D), lambda i: (0, 0))
  out_shape = [jax.ShapeDtypeStruct((_T, D), jnp.float32)] * 2
  if slab == 0:
    return pl.pallas_call(
        _cell_math,
        grid=(nblk,),
        in_specs=[row_in] * 4 + [wspec, wspec, bspec],
        out_specs=[row_out, row_out],
        out_shape=out_shape,
    )(hl, hr, cl, cr, Wl, Wr, b2d)
  anyspec = pl.BlockSpec(memory_space=pl.ANY)
  return pl.pallas_call(
      _tc_body_alias,
      grid=(nblk,),
      in_specs=[anyspec, anyspec] + [row_in] * 4 + [wspec, wspec, bspec],
      out_specs=[row_out, row_out],
      out_shape=out_shape,
      input_output_aliases={0: 0, 1: 1},
  )(h_prev, c_prev, hl, hr, cl, cr, Wl, Wr, b2d)


def kernel(h_bot, c_bot, h_buf, c_buf, Wl, Wr, b, bot_froms, prev_froms):
  idx_all = jnp.concatenate([
      jnp.asarray(bot_froms, jnp.int32),
      jnp.asarray(prev_froms, jnp.int32),
      jnp.broadcast_to(jnp.arange(_TPAD - _T, dtype=jnp.int32),
                       (2, _TPAD - _T)),
  ], axis=1)
  b2d = jnp.broadcast_to(b.astype(jnp.float32), (8, 5 * D))

  slabs = []
  for s in range(_NSLAB):
    il = lax.slice(idx_all[0], (s * _SROWS,), ((s + 1) * _SROWS,))
    ir = lax.slice(idx_all[1], (s * _SROWS,), ((s + 1) * _SROWS,))
    slabs.append(_sc_gather_slab(s, h_bot, c_bot, h_buf, c_buf, il, ir))

  h_acc = c_acc = None
  for s, (hl, cl, hr, cr) in enumerate(slabs):
    nblk = min(_SROWS, _T - s * _SROWS) // _BT
    h_acc, c_acc = _tc_slab(s, nblk, h_acc, c_acc, hl, hr, cl, cr,
                            Wl, Wr, b2d)
  return (h_acc, c_acc)


# speedup vs baseline: 15.2828x; 1.0191x over previous
import functools

import jax
import jax.numpy as jnp
from jax import lax
from jax.experimental import pallas as pl
from jax.experimental.pallas import tpu as pltpu
from jax.experimental.pallas import tpu_sc as plsc

D = 128
_LB = 60000
_LP = 40000
_T = _LB + _LP

_CH = 400
_NW = 32
_BT = 800

_BOT_SLABS = (30, 30, 30, 30, 30)
_BUF_SLABS = (26, 26, 24, 24)


def _sc_gather_slab(nch, tab_h, tab_c, idx_l, idx_r):
  rows = nch * _CH
  mesh = plsc.VectorSubcoreMesh(core_axis_name="c", subcore_axis_name="s")

  @functools.partial(
      pl.kernel, mesh=mesh,
      out_type=[jax.ShapeDtypeStruct((rows, D), jnp.float32)] * 4,
      scratch_types=[
          pltpu.VMEM((_CH,), jnp.int32),
          pltpu.VMEM((_CH, D), jnp.float32),
          pltpu.VMEM((_CH, D), jnp.float32),
          pltpu.SemaphoreType.DMA,
          pltpu.SemaphoreType.DMA,
      ],
  )
  def k(th, tc_, il, ir, hl_o, cl_o, hr_o, cr_o,
        idx_v, h_v, c_v, sem_h, sem_c):
    wid = lax.axis_index("s") * 2 + lax.axis_index("c")

    @pl.when(wid < nch)
    def _():
      base = wid * _CH
      for idx_hbm, h_o, c_o in ((il, hl_o, cl_o), (ir, hr_o, cr_o)):
        pltpu.sync_copy(idx_hbm.at[pl.ds(base, _CH)], idx_v)
        a = pltpu.async_copy(th.at[idx_v], h_v, sem_h)
        b = pltpu.async_copy(tc_.at[idx_v], c_v, sem_c)
        a.wait()
        b.wait()
        pltpu.sync_copy(h_v, h_o.at[pl.ds(base, _CH)])
        pltpu.sync_copy(c_v, c_o.at[pl.ds(base, _CH)])

  return k(tab_h, tab_c, idx_l, idx_r)


def _cell_math(hl_r, hr_r, cl_r, cr_r, wl_r, wr_r, b_r, h_o, c_o):
  g = jnp.dot(hl_r[...], wl_r[...], preferred_element_type=jnp.float32)
  g = g + jnp.dot(hr_r[...], wr_r[...], preferred_element_type=jnp.float32)
  g = g + b_r[0:1, :]
  i = jax.nn.sigmoid(g[:, 0:D])
  o = jax.nn.sigmoid(g[:, D:2 * D])
  u = jnp.tanh(g[:, 2 * D:3 * D])
  fl = jax.nn.sigmoid(g[:, 3 * D:4 * D])
  fr = jax.nn.sigmoid(g[:, 4 * D:5 * D])
  c = i * u + fl * cl_r[...] + fr * cr_r[...]
  h_o[...] = o * jnp.tanh(c)
  c_o[...] = c


def _tc_body_alias(hp, cp, hl_r, hr_r, cl_r, cr_r, wl_r, wr_r, b_r, h_o, c_o):
  _cell_math(hl_r, hr_r, cl_r, cr_r, wl_r, wr_r, b_r, h_o, c_o)


def _tc_slab(first, blk_base, nblk, h_prev, c_prev, hl, hr, cl, cr,
             Wl, Wr, b2d):
  row_in = pl.BlockSpec((_BT, D), lambda i: (i, 0))
  row_out = pl.BlockSpec((_BT, D), lambda i, _b=blk_base: (_b + i, 0))
  wspec = pl.BlockSpec((D, 5 * D), lambda i: (0, 0))
  bspec = pl.BlockSpec((8, 5 * D), lambda i: (0, 0))
  out_shape = [jax.ShapeDtypeStruct((_T, D), jnp.float32)] * 2
  if first:
    return pl.pallas_call(
        _cell_math,
        grid=(nblk,),
        in_specs=[row_in] * 4 + [wspec, wspec, bspec],
        out_specs=[row_out, row_out],
        out_shape=out_shape,
    )(hl, hr, cl, cr, Wl, Wr, b2d)
  anyspec = pl.BlockSpec(memory_space=pl.ANY)
  return pl.pallas_call(
      _tc_body_alias,
      grid=(nblk,),
      in_specs=[anyspec, anyspec] + [row_in] * 4 + [wspec, wspec, bspec],
      out_specs=[row_out, row_out],
      out_shape=out_shape,
      input_output_aliases={0: 0, 1: 1},
  )(h_prev, c_prev, hl, hr, cl, cr, Wl, Wr, b2d)


def kernel(h_bot, c_bot, h_buf, c_buf, Wl, Wr, b, bot_froms, prev_froms):
  bf = jnp.asarray(bot_froms, jnp.int32)
  pf = jnp.asarray(prev_froms, jnp.int32)
  b2d = jnp.broadcast_to(b.astype(jnp.float32), (8, 5 * D))

  plan = []
  off = 0
  for nch in _BOT_SLABS:
    rows = nch * _CH
    il = lax.slice(bf, (0, off), (1, off + rows)).reshape(rows)
    ir = lax.slice(bf, (1, off), (2, off + rows)).reshape(rows)
    plan.append((off, rows,
                 _sc_gather_slab(nch, h_bot, c_bot, il, ir)))
    off += rows
  offp = 0
  for nch in _BUF_SLABS:
    rows = nch * _CH
    il = lax.slice(pf, (0, offp), (1, offp + rows)).reshape(rows)
    ir = lax.slice(pf, (1, offp), (2, offp + rows)).reshape(rows)
    plan.append((_LB + offp, rows,
                 _sc_gather_slab(nch, h_buf, c_buf, il, ir)))
    offp += rows

  h_acc = c_acc = None
  for n, (row_base, rows, (hl, cl, hr, cr)) in enumerate(plan):
    h_acc, c_acc = _tc_slab(n == 0, row_base // _BT, rows // _BT,
                            h_acc, c_acc, hl, hr, cl, cr, Wl, Wr, b2d)
  return (h_acc, c_acc)
